# baseline (speedup 1.0000x reference)
import jax
import jax.numpy as jnp
from jax.experimental import pallas as pl


_B = 16384
_D = 64
_N = _B * _D // 128
_ROWS = 1024


def _dot_rows_kernel(x_ref, o_ref):
    p = x_ref[0] * x_ref[1]
    s0 = jnp.sum(p[:, :64], axis=1, keepdims=True)
    s1 = jnp.sum(p[:, 64:], axis=1, keepdims=True)
    o_ref[...] = jnp.concatenate([s0, s1], axis=1)


def kernel(inputs):
    x = inputs.reshape(2, _N, 128)
    out = pl.pallas_call(
        _dot_rows_kernel,
        grid=(_N // _ROWS,),
        in_specs=[pl.BlockSpec((2, _ROWS, 128), lambda i: (0, i, 0))],
        out_specs=pl.BlockSpec((_ROWS, 2), lambda i: (i, 0)),
        out_shape=jax.ShapeDtypeStruct((_N, 2), jnp.float32),
    )(x)
    return out.reshape(_B)


# speedup vs baseline: 1.8415x; 1.8415x over previous
import jax
import jax.numpy as jnp
from jax.experimental import pallas as pl


_B = 16384
_D = 64
_ROWS = 2048


def _dot_rows_kernel(gu_ref, fi_ref, o_ref):
    p = gu_ref[0] * fi_ref[0]
    o_ref[...] = jnp.sum(p, axis=1, keepdims=True)


def kernel(inputs):
    out = pl.pallas_call(
        _dot_rows_kernel,
        grid=(_B // _ROWS,),
        in_specs=[
            pl.BlockSpec((1, _ROWS, _D), lambda i: (0, i, 0)),
            pl.BlockSpec((1, _ROWS, _D), lambda i: (1, i, 0)),
        ],
        out_specs=pl.BlockSpec((_ROWS, 1), lambda i: (i, 0)),
        out_shape=jax.ShapeDtypeStruct((_B, 1), jnp.float32),
    )(inputs, inputs)
    return out.reshape(_B)


# speedup vs baseline: 1.8813x; 1.0216x over previous
import jax
import jax.numpy as jnp
from jax.experimental import pallas as pl


_B = 16384
_D = 64
_ROWS = 16384


def _dot_rows_kernel(gu_ref, fi_ref, o_ref):
    p = gu_ref[0] * fi_ref[0]
    o_ref[...] = jnp.sum(p, axis=1, keepdims=True)


def kernel(inputs):
    out = pl.pallas_call(
        _dot_rows_kernel,
        grid=(_B // _ROWS,),
        in_specs=[
            pl.BlockSpec((1, _ROWS, _D), lambda i: (0, i, 0)),
            pl.BlockSpec((1, _ROWS, _D), lambda i: (1, i, 0)),
        ],
        out_specs=pl.BlockSpec((_ROWS, 1), lambda i: (i, 0)),
        out_shape=jax.ShapeDtypeStruct((_B, 1), jnp.float32),
    )(inputs, inputs)
    return out.reshape(_B)


# speedup vs baseline: 8.0675x; 4.2882x over previous
import jax
import jax.numpy as jnp
from jax.experimental import pallas as pl


_B = 16384
_D = 64
_COLS = 2048


def _dot_cols_kernel(gu_ref, fi_ref, o_ref):
    p = gu_ref[0] * fi_ref[0]
    o_ref[...] = jnp.sum(p, axis=0)


def kernel(inputs):
    xt = jnp.transpose(inputs, (0, 2, 1))
    out = pl.pallas_call(
        _dot_cols_kernel,
        grid=(_B // _COLS,),
        in_specs=[
            pl.BlockSpec((1, _D, _COLS), lambda i: (0, 0, i)),
            pl.BlockSpec((1, _D, _COLS), lambda i: (1, 0, i)),
        ],
        out_specs=pl.BlockSpec((_COLS,), lambda i: (i,)),
        out_shape=jax.ShapeDtypeStruct((_B,), jnp.float32),
    )(xt, xt)
    return out


# speedup vs baseline: 8.3046x; 1.0294x over previous
import jax
import jax.numpy as jnp
from jax.experimental import pallas as pl
from jax.experimental.pallas import tpu as pltpu


_B = 16384
_D = 64
_NCHUNK = 8
_CH = _B // _NCHUNK


def _stream_dot_kernel(x_hbm, o_ref, bufs, sems):

    def start(c, slot):
        for p in range(2):
            pltpu.make_async_copy(
                x_hbm.at[p, :, pl.ds(c * _CH, _CH)],
                bufs.at[slot, p],
                sems.at[slot, p],
            ).start()

    def finish(c, slot):
        for p in range(2):
            pltpu.make_async_copy(
                x_hbm.at[p, :, pl.ds(c * _CH, _CH)],
                bufs.at[slot, p],
                sems.at[slot, p],
            ).wait()
        prod = bufs[slot, 0] * bufs[slot, 1]
        o_ref[pl.ds(c * _CH, _CH)] = jnp.sum(prod, axis=0)

    start(0, 0)
    start(1, 1)

    def body(g, _):
        c = 2 * g
        finish(c, 0)

        @pl.when(c + 2 < _NCHUNK)
        def _():
            start(c + 2, 0)

        finish(c + 1, 1)

        @pl.when(c + 3 < _NCHUNK)
        def _():
            start(c + 3, 1)

        return 0

    jax.lax.fori_loop(0, _NCHUNK // 2, body, 0)


def kernel(inputs):
    xt = jnp.transpose(inputs, (0, 2, 1))
    return pl.pallas_call(
        _stream_dot_kernel,
        in_specs=[pl.BlockSpec(memory_space=pltpu.MemorySpace.HBM)],
        out_specs=pl.BlockSpec(memory_space=pltpu.VMEM),
        out_shape=jax.ShapeDtypeStruct((_B,), jnp.float32),
        scratch_shapes=[
            pltpu.VMEM((2, 2, _D, _CH), jnp.float32),
            pltpu.SemaphoreType.DMA((2, 2)),
        ],
    )(xt)


# speedup vs baseline: 12.6631x; 1.5248x over previous
import jax
import jax.numpy as jnp
from jax.experimental import pallas as pl
from jax.experimental.pallas import tpu as pltpu


_B = 16384
_D = 64
_NCHUNK = 8
_CH = _B // _NCHUNK


def _stream_dot_kernel(x_hbm, o_ref, bufs, sems):

    def copy(c):
        return pltpu.make_async_copy(
            x_hbm.at[:, :, pl.ds(c * _CH, _CH)],
            bufs.at[c],
            sems.at[c],
        )

    for c in range(_NCHUNK):
        copy(c).start()
    for c in range(_NCHUNK):
        copy(c).wait()
        prod = bufs[c, 0] * bufs[c, 1]
        o_ref[pl.ds(c * _CH, _CH)] = jnp.sum(prod, axis=0)


def kernel(inputs):
    xt = jnp.transpose(inputs, (0, 2, 1))
    return pl.pallas_call(
        _stream_dot_kernel,
        in_specs=[pl.BlockSpec(memory_space=pltpu.MemorySpace.HBM)],
        out_specs=pl.BlockSpec(memory_space=pltpu.VMEM),
        out_shape=jax.ShapeDtypeStruct((_B,), jnp.float32),
        scratch_shapes=[
            pltpu.VMEM((_NCHUNK, 2, _D, _CH), jnp.float32),
            pltpu.SemaphoreType.DMA((_NCHUNK,)),
        ],
    )(xt)
